# Initial kernel scaffold; baseline (speedup 1.0000x reference)
import jax
import jax.numpy as jnp
from jax.experimental import pallas as pl


def kernel(slot_card_rows, slot_occupied, slot_tapped, game_info, card_table, W, b, zone_emb, slot_emb, empty_slot, tapped_vec, zone_ids, slot_ids):
    raise NotImplementedError("write your pallas kernel here")


# speedup vs baseline: 2.2211x; 2.2211x over previous
import functools

import jax
import jax.numpy as jnp
from jax import lax
from jax.experimental import pallas as pl
from jax.experimental.pallas import tpu as pltpu
from jax.experimental.pallas import tpu_sc as plsc

ZONE_SLOTS = 50
B_TOTAL = 4096
RAW = 128
DM = 128
GI_DIM = 90
OUT_DIM = ZONE_SLOTS * DM + GI_DIM

NUM_WORKERS = 32
ROWS_PER_WORKER = (B_TOTAL * ZONE_SLOTS) // NUM_WORKERS
CHUNK = 128
CHUNKS_PER_WORKER = ROWS_PER_WORKER // CHUNK


def _sc_gather(idx, table):
  n_rows = idx.shape[0] * idx.shape[1] * idx.shape[2]
  mesh = plsc.VectorSubcoreMesh(core_axis_name="c", subcore_axis_name="s")

  @functools.partial(
      pl.kernel,
      out_type=jax.ShapeDtypeStruct((n_rows, RAW), jnp.float32),
      mesh=mesh,
      scratch_types=[
          pltpu.VMEM((CHUNKS_PER_WORKER, CHUNK), jnp.int32),
          pltpu.VMEM((CHUNK, RAW), jnp.float32),
          pltpu.SemaphoreType.DMA,
      ],
  )
  def k(idx_hbm, table_hbm, out_hbm, idx_v, rows_v, sem):
    wid = lax.axis_index("s") * 2 + lax.axis_index("c")
    base_chunk = wid * CHUNKS_PER_WORKER
    pltpu.sync_copy(idx_hbm.at[wid], idx_v)

    def body(j, carry):
      pltpu.async_copy(table_hbm.at[idx_v.at[j]], rows_v, sem).wait()
      pltpu.sync_copy(
          rows_v, out_hbm.at[pl.ds((base_chunk + j) * CHUNK, CHUNK)])
      return carry

    lax.fori_loop(0, CHUNKS_PER_WORKER, body, 0)

  return k(idx, table)


def _tc_body(raw_ref, occ_ref, tap_ref, gi_ref, w_ref, b_ref, tv_ref,
             es_ref, pos_ref, out_ref):
  s = pl.program_id(1)

  @pl.when(s < ZONE_SLOTS)
  def _():
    rawm = raw_ref[0]
    proj = lax.dot_general(
        rawm, w_ref[...], (((1,), (1,)), ((), ())),
        preferred_element_type=jnp.float32) + b_ref[...]
    oh = (lax.broadcasted_iota(jnp.int32, (ZONE_SLOTS, 1), 0)
          == s).astype(jnp.float32)
    occ = lax.dot_general(occ_ref[...], oh, (((1,), (0,)), ((), ())),
                          preferred_element_type=jnp.float32)
    tap = lax.dot_general(tap_ref[...], oh, (((1,), (0,)), ((), ())),
                          preferred_element_type=jnp.float32)
    sv = (occ * (proj + tap * tv_ref[...])
          + (1.0 - occ) * es_ref[...] + pos_ref[0])
    out_ref[...] = sv

  @pl.when(s == ZONE_SLOTS)
  def _():
    out_ref[:, :GI_DIM] = gi_ref[...]


def kernel(slot_card_rows, slot_occupied, slot_tapped, game_info,
           card_table, W, b, zone_emb, slot_emb, empty_slot, tapped_vec,
           zone_ids, slot_ids):
  idx = slot_card_rows.astype(jnp.int32).T.reshape(
      NUM_WORKERS, CHUNKS_PER_WORKER, CHUNK)
  raw_g = _sc_gather(idx, card_table)
  raw_g = raw_g.reshape(ZONE_SLOTS, B_TOTAL, RAW)

  pos = (jnp.take(zone_emb, zone_ids, axis=0)
         + jnp.take(slot_emb, slot_ids, axis=0))
  pos = pos.reshape(ZONE_SLOTS, 1, DM)

  BB = 1024
  nb = B_TOTAL // BB
  grid = (nb, ZONE_SLOTS + 1)
  sclamp = lambda s: jnp.minimum(s, ZONE_SLOTS - 1)
  out = pl.pallas_call(
      _tc_body,
      grid=grid,
      in_specs=[
          pl.BlockSpec((1, BB, RAW), lambda i, s: (sclamp(s), i, 0)),
          pl.BlockSpec((BB, ZONE_SLOTS), lambda i, s: (i, 0)),
          pl.BlockSpec((BB, ZONE_SLOTS), lambda i, s: (i, 0)),
          pl.BlockSpec((BB, GI_DIM), lambda i, s: (i, 0)),
          pl.BlockSpec((DM, RAW), lambda i, s: (0, 0)),
          pl.BlockSpec((1, DM), lambda i, s: (0, 0)),
          pl.BlockSpec((1, DM), lambda i, s: (0, 0)),
          pl.BlockSpec((1, DM), lambda i, s: (0, 0)),
          pl.BlockSpec((1, 1, DM), lambda i, s: (sclamp(s), 0, 0)),
      ],
      out_specs=pl.BlockSpec((BB, DM), lambda i, s: (i, s)),
      out_shape=jax.ShapeDtypeStruct((B_TOTAL, OUT_DIM), jnp.float32),
  )(raw_g, slot_occupied, slot_tapped, game_info, W, b.reshape(1, DM),
    tapped_vec.reshape(1, DM), empty_slot.reshape(1, DM), pos)
  return out
